# baseline (speedup 1.0000x reference)
import functools

import jax
import jax.numpy as jnp
from jax import lax
from jax.experimental import pallas as pl
from jax.experimental.pallas import tpu as pltpu

_TINY = 1.1754943508222875e-38
_ONE_BITS = 0x3F800000
_KS1 = 42
_KS2 = 0x1BD11BDA ^ 42
_ROT_A = (13, 15, 26, 6)
_ROT_B = (17, 29, 16, 24)


def _rotl(v, r):
    return lax.shift_left(v, jnp.int32(r)) | lax.shift_right_logical(
        v, jnp.int32(32 - r))


def _four_rounds(x0, x1, rots):
    for r in rots:
        x0 = x0 + x1
        x1 = x0 ^ _rotl(x1, r)
    return x0, x1


def _threefry_bits(i):
    ks1 = jnp.int32(_KS1)
    ks2 = jnp.int32(_KS2)
    x0 = jnp.zeros_like(i)
    x1 = i + ks1
    x0, x1 = _four_rounds(x0, x1, _ROT_A)
    x0 = x0 + ks1
    x1 = x1 + (ks2 + jnp.int32(1))
    x0, x1 = _four_rounds(x0, x1, _ROT_B)
    x0 = x0 + ks2
    x1 = x1 + jnp.int32(2)
    x0, x1 = _four_rounds(x0, x1, _ROT_A)
    x1 = x1 + (ks1 + jnp.int32(3))
    x0, x1 = _four_rounds(x0, x1, _ROT_B)
    x0 = x0 + ks1
    x1 = x1 + (ks2 + jnp.int32(4))
    x0, x1 = _four_rounds(x0, x1, _ROT_A)
    x0 = x0 + ks2
    x1 = x1 + jnp.int32(5)
    return x0 ^ x1


def _gumbel_from_bits(bits):
    float_bits = lax.shift_right_logical(bits, jnp.int32(9)) | jnp.int32(
        _ONE_BITS)
    floats = lax.bitcast_convert_type(float_bits, jnp.float32) - jnp.float32(
        1.0)
    tiny = jnp.float32(_TINY)
    u = jnp.maximum(tiny, floats * (jnp.float32(1.0) - tiny) + tiny)
    return -jnp.log(-jnp.log(u))


def _body(x_ref, out_ref, vmax_ref, vidx_ref, *, rows, ncols, width, nblk):
    b = pl.program_id(0)

    @pl.when(b == 0)
    def _init():
        vmax_ref[...] = jnp.full((rows, 1), -jnp.inf, jnp.float32)
        vidx_ref[...] = jnp.zeros((rows, 1), jnp.int32)

    xb = x_ref[...]
    col = lax.broadcasted_iota(jnp.int32, (rows, width), 1) + b * width
    row = lax.broadcasted_iota(jnp.int32, (rows, width), 0)
    flat = row * ncols + col
    v = xb + _gumbel_from_bits(_threefry_bits(flat))
    v = jnp.where(col < ncols, v, -jnp.inf)
    m = jnp.max(v, axis=1, keepdims=True)
    idx = jnp.min(
        jnp.where(v == m, col, jnp.int32(0x7FFFFFFF)), axis=1, keepdims=True)
    upd = m > vmax_ref[...]
    vidx_ref[...] = jnp.where(upd, idx, vidx_ref[...])
    vmax_ref[...] = jnp.where(upd, m, vmax_ref[...])

    @pl.when(b == nblk - 1)
    def _fin():
        out_ref[...] = vidx_ref[...]


_WIDTH = 32768


@functools.partial(jax.jit, static_argnames=())
def kernel(x):
    rows, ncols = x.shape
    width = _WIDTH
    nblk = pl.cdiv(ncols, width)
    out = pl.pallas_call(
        functools.partial(
            _body, rows=rows, ncols=ncols, width=width, nblk=nblk),
        grid=(nblk,),
        in_specs=[pl.BlockSpec((rows, width), lambda b: (0, b))],
        out_specs=pl.BlockSpec((rows, 1), lambda b: (0, 0)),
        out_shape=jax.ShapeDtypeStruct((rows, 1), jnp.int32),
        scratch_shapes=[
            pltpu.VMEM((rows, 1), jnp.float32),
            pltpu.VMEM((rows, 1), jnp.int32),
        ],
    )(x)
    return out.reshape(rows)


# speedup vs baseline: 1.7079x; 1.7079x over previous
import functools

import jax
import jax.numpy as jnp
from jax import lax
from jax.experimental import pallas as pl
from jax.experimental.pallas import tpu as pltpu

_TINY = 1.1754943508222875e-38
_ONE_BITS = 0x3F800000
_KS1 = 42
_KS2 = 0x1BD11BDA ^ 42
_ROT_A = (13, 15, 26, 6)
_ROT_B = (17, 29, 16, 24)


def _rotl(v, r):
    return lax.shift_left(v, jnp.int32(r)) | lax.shift_right_logical(
        v, jnp.int32(32 - r))


def _four_rounds(x0, x1, rots):
    for r in rots:
        x0 = x0 + x1
        x1 = x0 ^ _rotl(x1, r)
    return x0, x1


def _threefry_bits(x1):
    ks1 = jnp.int32(_KS1)
    ks2 = jnp.int32(_KS2)
    x0 = x1
    x1 = x0 ^ _rotl(x1, _ROT_A[0])
    for r in _ROT_A[1:]:
        x0 = x0 + x1
        x1 = x0 ^ _rotl(x1, r)
    x0 = x0 + ks1
    x1 = x1 + (ks2 + jnp.int32(1))
    x0, x1 = _four_rounds(x0, x1, _ROT_B)
    x0 = x0 + ks2
    x1 = x1 + jnp.int32(2)
    x0, x1 = _four_rounds(x0, x1, _ROT_A)
    x1 = x1 + (ks1 + jnp.int32(3))
    x0, x1 = _four_rounds(x0, x1, _ROT_B)
    x0 = x0 + ks1
    x1 = x1 + (ks2 + jnp.int32(4))
    x0, x1 = _four_rounds(x0, x1, _ROT_A)
    x0 = x0 + ks2
    x1 = x1 + jnp.int32(5)
    return x0 ^ x1


def _gumbel_from_bits(bits):
    float_bits = lax.shift_right_logical(bits, jnp.int32(9)) | jnp.int32(
        _ONE_BITS)
    f = lax.bitcast_convert_type(float_bits, jnp.float32) - jnp.float32(1.0)
    u = f + jnp.float32(_TINY)
    return -jnp.log(-jnp.log(u))


_CHUNK = 256


def _body(x_ref, out_ref, bv_ref, bc_ref, *, rows, ncols, width, nblk):
    b = pl.program_id(0)
    nch = width // _CHUNK

    best_v = jnp.full((rows, _CHUNK), -jnp.inf, jnp.float32)
    best_c = jnp.zeros((rows, _CHUNK), jnp.int32)

    col0 = lax.broadcasted_iota(jnp.int32, (rows, _CHUNK), 1)
    row_term = lax.broadcasted_iota(jnp.int32, (rows, _CHUNK), 0) * ncols
    ctr0 = row_term + col0 + jnp.int32(_KS1)

    base = b * width
    for j in range(nch):
        off = j * _CHUNK
        xb = x_ref[:, off:off + _CHUNK]
        col = col0 + (base + off)
        v = xb + _gumbel_from_bits(_threefry_bits(ctr0 + (base + off)))
        v = jnp.where(col < ncols, v, -jnp.inf)
        upd = v > best_v
        best_v = jnp.where(upd, v, best_v)
        best_c = jnp.where(upd, col, best_c)

    @pl.when(b == 0)
    def _init():
        bv_ref[...] = best_v
        bc_ref[...] = best_c

    @pl.when(b > 0)
    def _merge():
        upd = best_v > bv_ref[...]
        bv_ref[...] = jnp.where(upd, best_v, bv_ref[...])
        bc_ref[...] = jnp.where(upd, best_c, bc_ref[...])

    @pl.when(b == nblk - 1)
    def _fin():
        m = jnp.max(bv_ref[...], axis=1, keepdims=True)
        idx = jnp.min(
            jnp.where(bv_ref[...] == m, bc_ref[...], jnp.int32(0x7FFFFFFF)),
            axis=1,
            keepdims=True)
        out_ref[...] = idx


_WIDTH = 8192


@functools.partial(jax.jit, static_argnames=())
def kernel(x):
    rows, ncols = x.shape
    width = _WIDTH
    nblk = pl.cdiv(ncols, width)
    out = pl.pallas_call(
        functools.partial(
            _body, rows=rows, ncols=ncols, width=width, nblk=nblk),
        grid=(nblk,),
        in_specs=[pl.BlockSpec((rows, width), lambda b: (0, b))],
        out_specs=pl.BlockSpec((rows, 1), lambda b: (0, 0)),
        out_shape=jax.ShapeDtypeStruct((rows, 1), jnp.int32),
        scratch_shapes=[
            pltpu.VMEM((rows, _CHUNK), jnp.float32),
            pltpu.VMEM((rows, _CHUNK), jnp.int32),
        ],
    )(x)
    return out.reshape(rows)
